# baseline (speedup 1.0000x reference)
import functools

import jax
import jax.numpy as jnp
from jax import lax
from jax.experimental import pallas as pl
from jax.experimental.pallas import tpu as pltpu
from jax.experimental.pallas import tpu_sc as plsc

D = 64
L = 16
NW = 32
SEQ = 200
BATCH = 1024
VOCAB = 1000000
B_PER_W = BATCH // NW
G_FULL = SEQ // L
TAIL = SEQ - L
SPLIT = 104
NBLK = VOCAB // 128
BLK_PER_W = -(-NBLK // NW)

_MESH = dict(core_axis_name="c", subcore_axis_name="s",
             num_cores=2, num_subcores=16)


def _transpose_body(tt_hbm, tail_hbm, pair_hbm, strip_v, blk_v, sem):
    wid = lax.axis_index("s") * 2 + lax.axis_index("c")
    iot = lax.iota(jnp.int32, L)

    def do_block(vcol, nrow):
        vcol = pl.multiple_of(vcol, 128)
        pltpu.sync_copy(tt_hbm.at[:, pl.ds(vcol, 2 * nrow)],
                        strip_v.at[:, pl.ds(0, 2 * nrow)])

        def prow(p, _):
            for e in range(2):
                col_vec = iot * 0 + (2 * p + e)
                for d0 in range(0, D, L):
                    g = plsc.load_gather(strip_v, [d0 + iot, col_vec])
                    blk_v[p, pl.ds(e * D + d0, L)] = g
            return 0

        lax.fori_loop(0, nrow, prow, 0)
        pltpu.sync_copy(blk_v.at[pl.ds(0, nrow)],
                        pair_hbm.at[pl.ds(pl.multiple_of(vcol // 2, 64), nrow)])

    def blk_body(k, _):
        blk = wid + k * NW

        @pl.when(blk < NBLK)
        def _():
            do_block(blk * 128, D)

        return 0

    lax.fori_loop(0, BLK_PER_W, blk_body, 0)

    @pl.when(wid == 0)
    def _():
        pltpu.sync_copy(tail_hbm, blk_v.at[pl.ds(0, 32)])
        pltpu.sync_copy(blk_v.at[pl.ds(0, 32)],
                        pair_hbm.at[pl.ds(NBLK * D, 32)])


def _gather_body(xi_hbm, tt_hbm, pair_hbm, seg_hbm, pe_hbm, out_hbm,
                 xi_v, tt_v, pidx_v, seg_v, add_v, pair_v, out_v, sem_rows):
    wid = lax.axis_index("s") * 2 + lax.axis_index("c")
    b0 = wid * B_PER_W

    pltpu.sync_copy(xi_hbm.at[pl.ds(b0, B_PER_W)], xi_v)
    pltpu.sync_copy(tt_hbm.at[pl.ds(b0, B_PER_W)], tt_v)
    pltpu.sync_copy(pe_hbm.at[pl.ds(0, SEQ)], out_v)
    pltpu.sync_copy(seg_hbm, seg_v)

    seg0_ = [seg_v[0, pl.ds(d * L, L)] for d in range(4)]
    seg1_ = [seg_v[1, pl.ds(d * L, L)] for d in range(4)]

    def build_add(s, _):
        for d in range(4):
            p = out_v[s, pl.ds(d * L, L)]
            add_v[s, pl.ds(d * L, L)] = p + seg0_[d]
            add_v[s, pl.ds(D + d * L, L)] = p + seg1_[d]
        return 0

    lax.fori_loop(0, SEQ, build_add, 0)

    group_offs = [g * L for g in range(G_FULL)] + [TAIL]

    def row_body(q, _):
        for off in group_offs:
            pidx_v[pl.ds(off, L)] = lax.shift_right_logical(
                xi_v[q, pl.ds(off, L)], 1)

        cp0 = pltpu.async_copy(
            pair_hbm.at[pidx_v.at[pl.ds(0, SPLIT)]],
            pair_v.at[pl.ds(0, SPLIT)], sem_rows)
        cp1 = pltpu.async_copy(
            pair_hbm.at[pidx_v.at[pl.ds(SPLIT, SEQ - SPLIT)]],
            pair_v.at[pl.ds(SPLIT, SEQ - SPLIT)], sem_rows)
        cp0.wait()
        cp1.wait()

        for off in group_offs:
            tvec = tt_v[q, pl.ds(off, L)]
            hvec = xi_v[q, pl.ds(off, L)]
            for r16 in range(L):
                r = off + r16
                toff = tvec[r16] * D
                half = (hvec[r16] & 1) * D
                for d in range(4):
                    out_v[r, pl.ds(d * L, L)] = (
                        pair_v[r, pl.ds(half + d * L, L)]
                        + add_v[r, pl.ds(toff + d * L, L)])

        pltpu.sync_copy(out_v, out_hbm.at[b0 + q])
        return 0

    lax.fori_loop(0, B_PER_W, row_body, 0)


@jax.jit
def _encode(xi, tt, table_t, tail2, segment_table, pe2d):
    transpose_k = pl.kernel(
        _transpose_body,
        out_type=jax.ShapeDtypeStruct((VOCAB // 2, 2 * D), jnp.float32),
        mesh=plsc.VectorSubcoreMesh(**_MESH),
        compiler_params=pltpu.CompilerParams(needs_layout_passes=False),
        scratch_types=[
            pltpu.VMEM((D, 2 * D), jnp.float32),
            pltpu.VMEM((D, 2 * D), jnp.float32),
            pltpu.SemaphoreType.DMA,
        ],
    )
    pair_table = transpose_k(table_t, tail2)

    gather_k = pl.kernel(
        _gather_body,
        out_type=jax.ShapeDtypeStruct((BATCH, SEQ, D), jnp.float32),
        mesh=plsc.VectorSubcoreMesh(**_MESH),
        scratch_types=[
            pltpu.VMEM((B_PER_W, SEQ), jnp.int32),
            pltpu.VMEM((B_PER_W, SEQ), jnp.int32),
            pltpu.VMEM((SEQ,), jnp.int32),
            pltpu.VMEM((2, D), jnp.float32),
            pltpu.VMEM((SEQ, 2 * D), jnp.float32),
            pltpu.VMEM((SEQ, 2 * D), jnp.float32),
            pltpu.VMEM((SEQ, D), jnp.float32),
            pltpu.SemaphoreType.DMA,
        ],
    )
    return gather_k(xi, tt, pair_table, segment_table, pe2d)


def kernel(x, token_types, token_table, segment_table, pe):
    xi = x.astype(jnp.int32)
    tt = token_types.astype(jnp.int32)
    pe2d = pe.reshape(pe.shape[-2], D)
    tail2 = token_table[NBLK * 128:].reshape(D // 2, 2 * D)
    return _encode(xi, tt, token_table.T, tail2, segment_table, pe2d)


# speedup vs baseline: 1.3734x; 1.3734x over previous
import functools

import jax
import jax.numpy as jnp
from jax import lax
from jax.experimental import pallas as pl
from jax.experimental.pallas import tpu as pltpu
from jax.experimental.pallas import tpu_sc as plsc

D = 64
L = 16
NW = 32
SEQ = 200
BATCH = 1024
VOCAB = 1000000
B_PER_W = BATCH // NW
G_FULL = SEQ // L
TAIL = SEQ - L
SPLIT = 104
NBLK = VOCAB // 128
BLK_PER_W = -(-NBLK // NW)

_MESH = dict(core_axis_name="c", subcore_axis_name="s",
             num_cores=2, num_subcores=16)


def _transpose_body(tt_hbm, tail_hbm, pair_hbm, strip_v, skew_v, blk_v, sem):
    wid = lax.axis_index("s") * 2 + lax.axis_index("c")
    iot = lax.iota(jnp.int32, L)
    SK = D + 1

    def do_block(vcol, nrow):
        vcol = pl.multiple_of(vcol, 128)
        pltpu.sync_copy(tt_hbm.at[:, pl.ds(vcol, 2 * nrow)],
                        strip_v.at[:, pl.ds(0, 2 * nrow)])

        def drow(d, _):
            for c0 in range(0, 2 * D, L):
                v = strip_v[d, pl.ds(c0, L)]
                plsc.store_scatter(skew_v, [(c0 + iot) * SK + d], v)
            return 0

        lax.fori_loop(0, D, drow, 0)

        def crow(c, _):
            for d0 in range(0, D, L):
                blk_v[c >> 1, pl.ds((c & 1) * D + d0, L)] = (
                    skew_v[pl.ds(c * SK + d0, L)])
            return 0

        lax.fori_loop(0, 2 * nrow, crow, 0)
        pltpu.sync_copy(blk_v.at[pl.ds(0, nrow)],
                        pair_hbm.at[pl.ds(pl.multiple_of(vcol // 2, 64), nrow)])

    def blk_body(k, _):
        blk = wid + k * NW

        @pl.when(blk < NBLK)
        def _():
            do_block(blk * 128, D)

        return 0

    lax.fori_loop(0, BLK_PER_W, blk_body, 0)

    @pl.when(wid == 0)
    def _():
        pltpu.sync_copy(tail_hbm, blk_v.at[pl.ds(0, 32)])
        pltpu.sync_copy(blk_v.at[pl.ds(0, 32)],
                        pair_hbm.at[pl.ds(NBLK * D, 32)])


def _gather_body(xi_hbm, tt_hbm, pair_hbm, seg_hbm, pe_hbm, out_hbm,
                 xi_v, tt_v, pidx_v, seg_v, add_v, pair_v, out_v, sem_rows):
    wid = lax.axis_index("s") * 2 + lax.axis_index("c")
    b0 = wid * B_PER_W

    pltpu.sync_copy(xi_hbm.at[pl.ds(b0, B_PER_W)], xi_v)
    pltpu.sync_copy(tt_hbm.at[pl.ds(b0, B_PER_W)], tt_v)
    pltpu.sync_copy(pe_hbm.at[pl.ds(0, SEQ)], out_v)
    pltpu.sync_copy(seg_hbm, seg_v)

    seg0_ = [seg_v[0, pl.ds(d * L, L)] for d in range(4)]
    seg1_ = [seg_v[1, pl.ds(d * L, L)] for d in range(4)]

    def build_add(s, _):
        for d in range(4):
            p = out_v[s, pl.ds(d * L, L)]
            add_v[s, pl.ds(d * L, L)] = p + seg0_[d]
            add_v[s, pl.ds(D + d * L, L)] = p + seg1_[d]
        return 0

    lax.fori_loop(0, SEQ, build_add, 0)

    group_offs = [g * L for g in range(G_FULL)] + [TAIL]

    def row_body(q, _):
        for off in group_offs:
            pidx_v[pl.ds(off, L)] = lax.shift_right_logical(
                xi_v[q, pl.ds(off, L)], 1)

        cp0 = pltpu.async_copy(
            pair_hbm.at[pidx_v.at[pl.ds(0, SPLIT)]],
            pair_v.at[pl.ds(0, SPLIT)], sem_rows)
        cp1 = pltpu.async_copy(
            pair_hbm.at[pidx_v.at[pl.ds(SPLIT, SEQ - SPLIT)]],
            pair_v.at[pl.ds(SPLIT, SEQ - SPLIT)], sem_rows)
        cp0.wait()
        cp1.wait()

        for off in group_offs:
            tvec = tt_v[q, pl.ds(off, L)]
            hvec = xi_v[q, pl.ds(off, L)]
            for r16 in range(L):
                r = off + r16
                toff = tvec[r16] * D
                half = (hvec[r16] & 1) * D
                for d in range(4):
                    out_v[r, pl.ds(d * L, L)] = (
                        pair_v[r, pl.ds(half + d * L, L)]
                        + add_v[r, pl.ds(toff + d * L, L)])

        pltpu.sync_copy(out_v, out_hbm.at[b0 + q])
        return 0

    lax.fori_loop(0, B_PER_W, row_body, 0)


@jax.jit
def _encode(xi, tt, table_t, tail2, segment_table, pe2d):
    transpose_k = pl.kernel(
        _transpose_body,
        out_type=jax.ShapeDtypeStruct((VOCAB // 2, 2 * D), jnp.float32),
        mesh=plsc.VectorSubcoreMesh(**_MESH),
        compiler_params=pltpu.CompilerParams(needs_layout_passes=False),
        scratch_types=[
            pltpu.VMEM((D, 2 * D), jnp.float32),
            pltpu.VMEM((2 * D * (D + 1),), jnp.float32),
            pltpu.VMEM((D, 2 * D), jnp.float32),
            pltpu.SemaphoreType.DMA,
        ],
    )
    pair_table = transpose_k(table_t, tail2)

    gather_k = pl.kernel(
        _gather_body,
        out_type=jax.ShapeDtypeStruct((BATCH, SEQ, D), jnp.float32),
        mesh=plsc.VectorSubcoreMesh(**_MESH),
        scratch_types=[
            pltpu.VMEM((B_PER_W, SEQ), jnp.int32),
            pltpu.VMEM((B_PER_W, SEQ), jnp.int32),
            pltpu.VMEM((SEQ,), jnp.int32),
            pltpu.VMEM((2, D), jnp.float32),
            pltpu.VMEM((SEQ, 2 * D), jnp.float32),
            pltpu.VMEM((SEQ, 2 * D), jnp.float32),
            pltpu.VMEM((SEQ, D), jnp.float32),
            pltpu.SemaphoreType.DMA,
        ],
    )
    return gather_k(xi, tt, pair_table, segment_table, pe2d)


def kernel(x, token_types, token_table, segment_table, pe):
    xi = x.astype(jnp.int32)
    tt = token_types.astype(jnp.int32)
    pe2d = pe.reshape(pe.shape[-2], D)
    tail2 = token_table[NBLK * 128:].reshape(D // 2, 2 * D)
    return _encode(xi, tt, token_table.T, tail2, segment_table, pe2d)


# speedup vs baseline: 1.3752x; 1.0013x over previous
import functools

import jax
import jax.numpy as jnp
from jax import lax
from jax.experimental import pallas as pl
from jax.experimental.pallas import tpu as pltpu
from jax.experimental.pallas import tpu_sc as plsc

D = 64
L = 16
NW = 32
SEQ = 200
BATCH = 1024
VOCAB = 1000000
B_PER_W = BATCH // NW
G_FULL = SEQ // L
TAIL = SEQ - L
SPLIT = 104
NBLK = VOCAB // 128
BLK_PER_W = -(-NBLK // NW)

_MESH = dict(core_axis_name="c", subcore_axis_name="s",
             num_cores=2, num_subcores=16)


def _transpose_body(tt_hbm, tail_hbm, pair_hbm, strip_v, skew_v, blk_v, sem):
    wid = lax.axis_index("s") * 2 + lax.axis_index("c")
    iot = lax.iota(jnp.int32, L)
    SK = D + 1

    def do_block(vcol, nrow):
        vcol = pl.multiple_of(vcol, 128)
        pltpu.sync_copy(tt_hbm.at[:, pl.ds(vcol, 2 * nrow)],
                        strip_v.at[:, pl.ds(0, 2 * nrow)])

        def drow(d4, _):
            for dd in range(4):
                d = d4 * 4 + dd
                for c0 in range(0, 2 * D, L):
                    v = strip_v[d, pl.ds(c0, L)]
                    plsc.store_scatter(skew_v, [(c0 + iot) * SK + d], v)
            return 0

        lax.fori_loop(0, D // 4, drow, 0)

        def crow(p, _):
            for e in range(2):
                base = (2 * p + e) * SK
                for d0 in range(0, D, L):
                    blk_v[p, pl.ds(e * D + d0, L)] = (
                        skew_v[pl.ds(base + d0, L)])
            return 0

        lax.fori_loop(0, nrow, crow, 0)
        pltpu.sync_copy(blk_v.at[pl.ds(0, nrow)],
                        pair_hbm.at[pl.ds(pl.multiple_of(vcol // 2, 64), nrow)])

    def blk_body(k, _):
        blk = wid + k * NW

        @pl.when(blk < NBLK)
        def _():
            do_block(blk * 128, D)

        return 0

    lax.fori_loop(0, BLK_PER_W, blk_body, 0)

    @pl.when(wid == 0)
    def _():
        pltpu.sync_copy(tail_hbm, blk_v.at[pl.ds(0, 32)])
        pltpu.sync_copy(blk_v.at[pl.ds(0, 32)],
                        pair_hbm.at[pl.ds(NBLK * D, 32)])


def _gather_body(xi_hbm, tt_hbm, pair_hbm, seg_hbm, pe_hbm, out_hbm,
                 xi_v, tt_v, pidx_v, seg_v, add_v, pair_v, out_v, sem_rows):
    wid = lax.axis_index("s") * 2 + lax.axis_index("c")
    b0 = wid * B_PER_W

    pltpu.sync_copy(xi_hbm.at[pl.ds(b0, B_PER_W)], xi_v)
    pltpu.sync_copy(tt_hbm.at[pl.ds(b0, B_PER_W)], tt_v)
    pltpu.sync_copy(pe_hbm.at[pl.ds(0, SEQ)], out_v)
    pltpu.sync_copy(seg_hbm, seg_v)

    seg0_ = [seg_v[0, pl.ds(d * L, L)] for d in range(4)]
    seg1_ = [seg_v[1, pl.ds(d * L, L)] for d in range(4)]

    def build_add(s, _):
        for d in range(4):
            p = out_v[s, pl.ds(d * L, L)]
            add_v[s, pl.ds(d * L, L)] = p + seg0_[d]
            add_v[s, pl.ds(D + d * L, L)] = p + seg1_[d]
        return 0

    lax.fori_loop(0, SEQ, build_add, 0)

    group_offs = [g * L for g in range(G_FULL)] + [TAIL]

    def row_body(q, _):
        for off in group_offs:
            pidx_v[pl.ds(off, L)] = lax.shift_right_logical(
                xi_v[q, pl.ds(off, L)], 1)

        cp0 = pltpu.async_copy(
            pair_hbm.at[pidx_v.at[pl.ds(0, SPLIT)]],
            pair_v.at[pl.ds(0, SPLIT)], sem_rows)
        cp1 = pltpu.async_copy(
            pair_hbm.at[pidx_v.at[pl.ds(SPLIT, SEQ - SPLIT)]],
            pair_v.at[pl.ds(SPLIT, SEQ - SPLIT)], sem_rows)
        cp0.wait()
        cp1.wait()

        for off in group_offs:
            tvec = tt_v[q, pl.ds(off, L)]
            hvec = xi_v[q, pl.ds(off, L)]
            for r16 in range(L):
                r = off + r16
                toff = tvec[r16] * D
                half = (hvec[r16] & 1) * D
                for d in range(4):
                    out_v[r, pl.ds(d * L, L)] = (
                        pair_v[r, pl.ds(half + d * L, L)]
                        + add_v[r, pl.ds(toff + d * L, L)])

        pltpu.sync_copy(out_v, out_hbm.at[b0 + q])
        return 0

    lax.fori_loop(0, B_PER_W, row_body, 0)


@jax.jit
def _encode(xi, tt, table_t, tail2, segment_table, pe2d):
    transpose_k = pl.kernel(
        _transpose_body,
        out_type=jax.ShapeDtypeStruct((VOCAB // 2, 2 * D), jnp.float32),
        mesh=plsc.VectorSubcoreMesh(**_MESH),
        compiler_params=pltpu.CompilerParams(needs_layout_passes=False),
        scratch_types=[
            pltpu.VMEM((D, 2 * D), jnp.float32),
            pltpu.VMEM((2 * D * (D + 1),), jnp.float32),
            pltpu.VMEM((D, 2 * D), jnp.float32),
            pltpu.SemaphoreType.DMA,
        ],
    )
    pair_table = transpose_k(table_t, tail2)

    gather_k = pl.kernel(
        _gather_body,
        out_type=jax.ShapeDtypeStruct((BATCH, SEQ, D), jnp.float32),
        mesh=plsc.VectorSubcoreMesh(**_MESH),
        scratch_types=[
            pltpu.VMEM((B_PER_W, SEQ), jnp.int32),
            pltpu.VMEM((B_PER_W, SEQ), jnp.int32),
            pltpu.VMEM((SEQ,), jnp.int32),
            pltpu.VMEM((2, D), jnp.float32),
            pltpu.VMEM((SEQ, 2 * D), jnp.float32),
            pltpu.VMEM((SEQ, 2 * D), jnp.float32),
            pltpu.VMEM((SEQ, D), jnp.float32),
            pltpu.SemaphoreType.DMA,
        ],
    )
    return gather_k(xi, tt, pair_table, segment_table, pe2d)


def kernel(x, token_types, token_table, segment_table, pe):
    xi = x.astype(jnp.int32)
    tt = token_types.astype(jnp.int32)
    pe2d = pe.reshape(pe.shape[-2], D)
    tail2 = token_table[NBLK * 128:].reshape(D // 2, 2 * D)
    return _encode(xi, tt, token_table.T, tail2, segment_table, pe2d)


# speedup vs baseline: 2.4417x; 1.7755x over previous
import functools

import jax
import jax.numpy as jnp
from jax import lax
from jax.experimental import pallas as pl
from jax.experimental.pallas import tpu as pltpu
from jax.experimental.pallas import tpu_sc as plsc

D = 64
L = 16
NW = 32
SEQ = 200
BATCH = 1024
B_PER_W = BATCH // NW
G_FULL = SEQ // L
TAIL = SEQ - L
SPLIT = 104


def _sc_body(xi_hbm, tt_hbm, table_hbm, seg_hbm, pe_hbm, out_hbm,
             xi_v, tt_v, seg_v, add_v, rows_v, out_v, sem0, sem1):
    wid = lax.axis_index("s") * 2 + lax.axis_index("c")
    b0 = wid * B_PER_W
    sems = [sem0, sem1]

    pltpu.sync_copy(xi_hbm.at[pl.ds(b0, B_PER_W)], xi_v)
    pltpu.sync_copy(tt_hbm.at[pl.ds(b0, B_PER_W)], tt_v)
    pltpu.sync_copy(pe_hbm.at[pl.ds(0, SEQ)], out_v.at[0])
    pltpu.sync_copy(seg_hbm, seg_v)

    seg0_ = [seg_v[0, pl.ds(d * L, L)] for d in range(4)]
    seg1_ = [seg_v[1, pl.ds(d * L, L)] for d in range(4)]

    def build_add(s, _):
        for d in range(4):
            p = out_v[0, s, pl.ds(d * L, L)]
            add_v[s, pl.ds(d * L, L)] = p + seg0_[d]
            add_v[s, pl.ds(D + d * L, L)] = p + seg1_[d]
        return 0

    lax.fori_loop(0, SEQ, build_add, 0)

    group_offs = [g * L for g in range(G_FULL)] + [TAIL]

    def issue_gather(q, buf, sem):
        cp0 = pltpu.async_copy(
            table_hbm.at[xi_v.at[q, pl.ds(0, SPLIT)]],
            rows_v.at[buf, pl.ds(0, SPLIT)], sem)
        cp1 = pltpu.async_copy(
            table_hbm.at[xi_v.at[q, pl.ds(SPLIT, SEQ - SPLIT)]],
            rows_v.at[buf, pl.ds(SPLIT, SEQ - SPLIT)], sem)
        return cp0, cp1

    def wait_gather(q, buf, sem):
        pltpu.make_async_copy(
            table_hbm.at[xi_v.at[q, pl.ds(0, SPLIT)]],
            rows_v.at[buf, pl.ds(0, SPLIT)], sem).wait()
        pltpu.make_async_copy(
            table_hbm.at[xi_v.at[q, pl.ds(SPLIT, SEQ - SPLIT)]],
            rows_v.at[buf, pl.ds(SPLIT, SEQ - SPLIT)], sem).wait()

    issue_gather(0, 0, sem0)

    def row_body(q, _):
        buf = q & 1

        @pl.when(jnp.logical_and(q + 1 < B_PER_W, buf == 0))
        def _():
            issue_gather(q + 1, 1, sem1)

        @pl.when(jnp.logical_and(q + 1 < B_PER_W, buf == 1))
        def _():
            issue_gather(q + 1, 0, sem0)

        @pl.when(buf == 0)
        def _():
            wait_gather(q, 0, sem0)

        @pl.when(buf == 1)
        def _():
            wait_gather(q, 1, sem1)

        for off in group_offs:
            tvec = tt_v[q, pl.ds(off, L)]
            for r16 in range(L):
                r = off + r16
                toff = tvec[r16] * D
                for d in range(4):
                    out_v[buf, r, pl.ds(d * L, L)] = (
                        rows_v[buf, r, pl.ds(d * L, L)]
                        + add_v[r, pl.ds(toff + d * L, L)])

        pltpu.sync_copy(out_v.at[buf], out_hbm.at[b0 + q])
        return 0

    lax.fori_loop(0, B_PER_W, row_body, 0)


@jax.jit
def _encode(xi, tt, table, segment_table, pe2d):
    mesh = plsc.VectorSubcoreMesh(
        core_axis_name="c", subcore_axis_name="s", num_cores=2, num_subcores=16)
    run = pl.kernel(
        _sc_body,
        out_type=jax.ShapeDtypeStruct((BATCH, SEQ, D), jnp.float32),
        mesh=mesh,
        compiler_params=pltpu.CompilerParams(use_tc_tiling_on_sc=False),
        scratch_types=[
            pltpu.VMEM((B_PER_W, SEQ), jnp.int32),
            pltpu.VMEM((B_PER_W, SEQ), jnp.int32),
            pltpu.VMEM((2, D), jnp.float32),
            pltpu.VMEM((SEQ, 2 * D), jnp.float32),
            pltpu.VMEM((2, SEQ, D), jnp.float32),
            pltpu.VMEM((2, SEQ, D), jnp.float32),
            pltpu.SemaphoreType.DMA,
            pltpu.SemaphoreType.DMA,
        ],
    )
    return run(xi, tt, table, segment_table, pe2d)


def kernel(x, token_types, token_table, segment_table, pe):
    xi = x.astype(jnp.int32)
    tt = token_types.astype(jnp.int32)
    pe2d = pe.reshape(pe.shape[-2], D)
    return _encode(xi, tt, token_table, segment_table, pe2d)


# speedup vs baseline: 2.5676x; 1.0516x over previous
import functools

import jax
import jax.numpy as jnp
from jax import lax
from jax.experimental import pallas as pl
from jax.experimental.pallas import tpu as pltpu
from jax.experimental.pallas import tpu_sc as plsc

D = 64
L = 16
NW = 32
SEQ = 200
BATCH = 1024
B_PER_W = BATCH // NW
G_FULL = SEQ // L
TAIL = SEQ - L
SPLIT = 104


def _sc_body(xi_hbm, tt_hbm, table_hbm, seg_hbm, pe_hbm, out_hbm,
             xi_v, tt_v, pidx_v, seg_v, add_v, rows_v, out_v, sem0, sem1):
    wid = lax.axis_index("s") * 2 + lax.axis_index("c")
    b0 = wid * B_PER_W
    sems = [sem0, sem1]

    pltpu.sync_copy(xi_hbm.at[pl.ds(b0, B_PER_W)], xi_v)
    pltpu.sync_copy(tt_hbm.at[pl.ds(b0, B_PER_W)], tt_v)
    pltpu.sync_copy(pe_hbm.at[pl.ds(0, SEQ)], out_v)
    pltpu.sync_copy(seg_hbm, seg_v)

    group_offs = [g * L for g in range(G_FULL)] + [TAIL]

    seg0_ = [seg_v[0, pl.ds(d * L, L)] for d in range(4)]
    seg1_ = [seg_v[1, pl.ds(d * L, L)] for d in range(4)]

    def build_add(s, _):
        for d in range(4):
            p = out_v[s, pl.ds(d * L, L)]
            add_v[s, pl.ds(d * L, L)] = p + seg0_[d]
            add_v[s, pl.ds(D + d * L, L)] = p + seg1_[d]
        return 0

    lax.fori_loop(0, SEQ, build_add, 0)

    def issue_gather(q, buf, sem):
        pb = buf * SEQ
        for off in group_offs:
            pidx_v[pl.ds(pb + off, L)] = xi_v[q, pl.ds(off, L)]
        cp0 = pltpu.async_copy(
            table_hbm.at[pidx_v.at[pl.ds(pb, SPLIT)]],
            rows_v.at[buf, pl.ds(0, SPLIT)], sem)
        cp1 = pltpu.async_copy(
            table_hbm.at[pidx_v.at[pl.ds(pb + SPLIT, SEQ - SPLIT)]],
            rows_v.at[buf, pl.ds(SPLIT, SEQ - SPLIT)], sem)
        return cp0, cp1

    def wait_gather(q, buf, sem):
        pb = buf * SEQ
        pltpu.make_async_copy(
            table_hbm.at[pidx_v.at[pl.ds(pb, SPLIT)]],
            rows_v.at[buf, pl.ds(0, SPLIT)], sem).wait()
        pltpu.make_async_copy(
            table_hbm.at[pidx_v.at[pl.ds(pb + SPLIT, SEQ - SPLIT)]],
            rows_v.at[buf, pl.ds(SPLIT, SEQ - SPLIT)], sem).wait()

    issue_gather(0, 0, sem0)

    def row_body(q, _):
        buf = q & 1

        @pl.when(jnp.logical_and(q + 1 < B_PER_W, buf == 0))
        def _():
            issue_gather(q + 1, 1, sem1)

        @pl.when(jnp.logical_and(q + 1 < B_PER_W, buf == 1))
        def _():
            issue_gather(q + 1, 0, sem0)

        @pl.when(buf == 0)
        def _():
            wait_gather(q, 0, sem0)

        @pl.when(buf == 1)
        def _():
            wait_gather(q, 1, sem1)

        for off in group_offs:
            tvec = tt_v[q, pl.ds(off, L)]
            for r16 in range(L):
                r = off + r16
                toff = tvec[r16] * D
                for d in range(4):
                    out_v[r, pl.ds(d * L, L)] = (
                        rows_v[buf, r, pl.ds(d * L, L)]
                        + add_v[r, pl.ds(toff + d * L, L)])

        pltpu.sync_copy(out_v, out_hbm.at[b0 + q])
        return 0

    lax.fori_loop(0, B_PER_W, row_body, 0)


@jax.jit
def _encode(xi, tt, table, segment_table, pe2d):
    mesh = plsc.VectorSubcoreMesh(
        core_axis_name="c", subcore_axis_name="s", num_cores=2, num_subcores=16)
    run = pl.kernel(
        _sc_body,
        out_type=jax.ShapeDtypeStruct((BATCH, SEQ, D), jnp.float32),
        mesh=mesh,
        scratch_types=[
            pltpu.VMEM((B_PER_W, SEQ), jnp.int32),
            pltpu.VMEM((B_PER_W, SEQ), jnp.int32),
            pltpu.VMEM((2 * SEQ,), jnp.int32),
            pltpu.VMEM((8, D), jnp.float32),
            pltpu.VMEM((SEQ, 2 * D), jnp.float32),
            pltpu.VMEM((2, SEQ, 2 * D), jnp.float32),
            pltpu.VMEM((SEQ, D), jnp.float32),
            pltpu.SemaphoreType.DMA,
            pltpu.SemaphoreType.DMA,
        ],
    )
    return run(xi, tt, table, segment_table, pe2d)


def kernel(x, token_types, token_table, segment_table, pe):
    xi = x.astype(jnp.int32)
    tt = token_types.astype(jnp.int32)
    pe2d = pe.reshape(pe.shape[-2], D)
    tablep = jnp.pad(token_table, ((0, 0), (0, D)))
    seg8 = jnp.pad(segment_table, ((0, 6), (0, 0)))
    return _encode(xi, tt, tablep, seg8, pe2d)
